# Initial kernel scaffold; baseline (speedup 1.0000x reference)
import jax
import jax.numpy as jnp
from jax.experimental import pallas as pl


def kernel(factored_tokens, W0, W1, W2):
    raise NotImplementedError("write your pallas kernel here")


# speedup vs baseline: 4.9712x; 4.9712x over previous
import functools

import jax
import jax.numpy as jnp
from jax import lax
from jax.experimental import pallas as pl
from jax.experimental.pallas import tpu as pltpu
from jax.experimental.pallas import tpu_sc as plsc

B, L, D = 4096, 200, 64
N = B * L
NC, NS = 2, 16
NW = NC * NS
STEP = 128
TPW = N // NW
NSTEPS = TPW // STEP
NROWS = N // STEP


def _emb_body(idx0_hbm, idx1_hbm, idx2_hbm, w0_hbm, w1_hbm, w2_hbm,
              out_hbm, idx0_v, idx1_v, idx2_v, rows0, rows1, rows2, sem):
    cid = lax.axis_index("c")
    sid = lax.axis_index("s")
    wid = sid * NC + cid
    rbase = wid * NSTEPS
    obase = wid * TPW

    pltpu.sync_copy(idx0_hbm.at[pl.ds(rbase, NSTEPS)], idx0_v)
    pltpu.sync_copy(idx1_hbm.at[pl.ds(rbase, NSTEPS)], idx1_v)
    pltpu.sync_copy(idx2_hbm.at[pl.ds(rbase, NSTEPS)], idx2_v)

    def step(t, carry):
        c0 = pltpu.async_copy(w0_hbm.at[idx0_v.at[t]], rows0, sem)
        c1 = pltpu.async_copy(w1_hbm.at[idx1_v.at[t]], rows1, sem)
        c2 = pltpu.async_copy(w2_hbm.at[idx2_v.at[t]], rows2, sem)
        c0.wait()
        c1.wait()
        c2.wait()

        def addrow(i, c):
            for k in range(D // 16):
                sl = pl.ds(k * 16, 16)
                rows0[i, sl] = rows0[i, sl] + rows1[i, sl] + rows2[i, sl]
            return c

        lax.fori_loop(0, STEP, addrow, 0, unroll=2)
        pltpu.sync_copy(rows0, out_hbm.at[pl.ds(obase + t * STEP, STEP)])
        return carry

    lax.fori_loop(0, NSTEPS, step, 0)


@functools.partial(jax.jit, static_argnames=())
def _emb_call(idx0, idx1, idx2, W0, W1, W2):
    mesh = plsc.VectorSubcoreMesh(core_axis_name="c", subcore_axis_name="s")
    return pl.kernel(
        _emb_body,
        out_type=jax.ShapeDtypeStruct((N, D), jnp.float32),
        mesh=mesh,
        scratch_types=[
            pltpu.VMEM((NSTEPS, STEP), jnp.int32),
            pltpu.VMEM((NSTEPS, STEP), jnp.int32),
            pltpu.VMEM((NSTEPS, STEP), jnp.int32),
            pltpu.VMEM((STEP, D), jnp.float32),
            pltpu.VMEM((STEP, D), jnp.float32),
            pltpu.VMEM((STEP, D), jnp.float32),
            pltpu.SemaphoreType.DMA,
        ],
        compiler_params=pltpu.CompilerParams(use_tc_tiling_on_sc=False),
    )(idx0, idx1, idx2, W0, W1, W2)


def kernel(factored_tokens, W0, W1, W2):
    ft = factored_tokens.reshape(N, 3).astype(jnp.int32)
    idx = ft.T.reshape(3, NROWS, STEP)
    out = _emb_call(idx[0], idx[1], idx[2], W0, W1, W2)
    return out.reshape(B, L, D)


# speedup vs baseline: 8.1847x; 1.6464x over previous
import functools

import jax
import jax.numpy as jnp
from jax import lax
from jax.experimental import pallas as pl
from jax.experimental.pallas import tpu as pltpu
from jax.experimental.pallas import tpu_sc as plsc

B, L, D = 4096, 200, 64
N = B * L
NC, NS = 2, 16
NW = NC * NS
STEP = 128
TPW = N // NW
NSTEPS = TPW // STEP
NROWS = N // STEP


def _emb_body(idx0_hbm, idx1_hbm, idx2_hbm, w0_hbm, w1_hbm, w2_hbm,
              out_hbm, idx0_v, idx1_v, idx2_v,
              r0a, r0b, r0c, r1a, r1b, r1c,
              sem_in_e, sem_in_o, sem_out_e, sem_out_o):
    cid = lax.axis_index("c")
    sid = lax.axis_index("s")
    wid = sid * NC + cid
    rbase = wid * NSTEPS
    obase = wid * TPW

    pltpu.sync_copy(idx0_hbm.at[pl.ds(rbase, NSTEPS)], idx0_v)
    pltpu.sync_copy(idx1_hbm.at[pl.ds(rbase, NSTEPS)], idx1_v)
    pltpu.sync_copy(idx2_hbm.at[pl.ds(rbase, NSTEPS)], idx2_v)

    def fire(t, ra, rb, rc, sem):
        pltpu.async_copy(w0_hbm.at[idx0_v.at[t]], ra, sem)
        pltpu.async_copy(w1_hbm.at[idx1_v.at[t]], rb, sem)
        pltpu.async_copy(w2_hbm.at[idx2_v.at[t]], rc, sem)

    def drain_in(t, ra, rb, rc, sem):
        pltpu.make_async_copy(w0_hbm.at[idx0_v.at[t]], ra, sem).wait()
        pltpu.make_async_copy(w1_hbm.at[idx1_v.at[t]], rb, sem).wait()
        pltpu.make_async_copy(w2_hbm.at[idx2_v.at[t]], rc, sem).wait()

    def accum(ra, rb, rc):
        def addrow(i, c):
            for k in range(D // 16):
                sl = pl.ds(k * 16, 16)
                plsc.addupdate(ra.at[i, sl], rb[i, sl] + rc[i, sl])
            return c
        lax.fori_loop(0, STEP, addrow, 0, unroll=2)

    def fire_out(t, ra, sem):
        pltpu.async_copy(ra, out_hbm.at[pl.ds(obase + t * STEP, STEP)], sem)

    def drain_out(ra, sem):
        pltpu.make_async_copy(ra, out_hbm.at[pl.ds(obase, STEP)], sem).wait()

    fire(0, r0a, r0b, r0c, sem_in_e)

    def body2(u, carry):
        t0 = 2 * u
        t1 = t0 + 1

        @pl.when(u > 0)
        def _():
            drain_out(r1a, sem_out_o)
        fire(t1, r1a, r1b, r1c, sem_in_o)

        drain_in(t0, r0a, r0b, r0c, sem_in_e)
        accum(r0a, r0b, r0c)
        fire_out(t0, r0a, sem_out_e)

        @pl.when(u < NSTEPS // 2 - 1)
        def _():
            drain_out(r0a, sem_out_e)
            fire(t0 + 2, r0a, r0b, r0c, sem_in_e)

        drain_in(t1, r1a, r1b, r1c, sem_in_o)
        accum(r1a, r1b, r1c)
        fire_out(t1, r1a, sem_out_o)
        return carry

    lax.fori_loop(0, NSTEPS // 2, body2, 0)
    drain_out(r0a, sem_out_e)
    drain_out(r1a, sem_out_o)


@functools.partial(jax.jit, static_argnames=())
def _emb_call(idx0, idx1, idx2, W0, W1, W2):
    mesh = plsc.VectorSubcoreMesh(core_axis_name="c", subcore_axis_name="s")
    return pl.kernel(
        _emb_body,
        out_type=jax.ShapeDtypeStruct((N, D), jnp.float32),
        mesh=mesh,
        scratch_types=[
            pltpu.VMEM((NSTEPS, STEP), jnp.int32),
            pltpu.VMEM((NSTEPS, STEP), jnp.int32),
            pltpu.VMEM((NSTEPS, STEP), jnp.int32),
            pltpu.VMEM((STEP, D), jnp.float32),
            pltpu.VMEM((STEP, D), jnp.float32),
            pltpu.VMEM((STEP, D), jnp.float32),
            pltpu.VMEM((STEP, D), jnp.float32),
            pltpu.VMEM((STEP, D), jnp.float32),
            pltpu.VMEM((STEP, D), jnp.float32),
            pltpu.SemaphoreType.DMA,
            pltpu.SemaphoreType.DMA,
            pltpu.SemaphoreType.DMA,
            pltpu.SemaphoreType.DMA,
        ],
        compiler_params=pltpu.CompilerParams(use_tc_tiling_on_sc=False),
    )(idx0, idx1, idx2, W0, W1, W2)


def kernel(factored_tokens, W0, W1, W2):
    ft = factored_tokens.reshape(N, 3).astype(jnp.int32)
    idx = ft.T.reshape(3, NROWS, STEP)
    out = _emb_call(idx[0], idx[1], idx[2], W0, W1, W2)
    return out.reshape(B, L, D)


# speedup vs baseline: 8.5009x; 1.0386x over previous
import functools

import jax
import jax.numpy as jnp
from jax import lax
from jax.experimental import pallas as pl
from jax.experimental.pallas import tpu as pltpu
from jax.experimental.pallas import tpu_sc as plsc

B, L, D = 4096, 200, 64
N = B * L
NC, NS = 2, 16
NW = NC * NS
STEP = 128
TPW = N // NW
NSTEPS = TPW // STEP
NROWS = N // STEP
NBUF = 4


def _emb_body(idx0_hbm, idx1_hbm, idx2_hbm, w0_hbm, w1_hbm, w2_hbm,
              out_hbm, idx_v, rows, sem_idx, sem_in, sem_out):
    cid = lax.axis_index("c")
    sid = lax.axis_index("s")
    wid = sid * NC + cid
    rbase = wid * NSTEPS
    obase = wid * TPW
    whbm = (w0_hbm, w1_hbm, w2_hbm)
    ihbm = (idx0_hbm, idx1_hbm, idx2_hbm)

    def fire_idx(t, b):
        for f in range(3):
            pltpu.async_copy(ihbm[f].at[rbase + t], idx_v.at[b, f],
                             sem_idx.at[b])

    def drain_idx(b):
        for f in range(3):
            pltpu.make_async_copy(ihbm[f].at[rbase], idx_v.at[b, f],
                                  sem_idx.at[b]).wait()

    def fire_gather(b):
        for f in range(3):
            pltpu.async_copy(whbm[f].at[idx_v.at[b, f]], rows.at[b, f],
                             sem_in.at[b])

    def drain_gather(b):
        for f in range(3):
            pltpu.make_async_copy(whbm[f].at[idx_v.at[b, f]], rows.at[b, f],
                                  sem_in.at[b]).wait()

    def accum(b):
        def addrow(i, c):
            for k in range(D // 16):
                sl = pl.ds(k * 16, 16)
                plsc.addupdate(rows.at[b, 0, i, sl],
                               rows[b, 1, i, sl] + rows[b, 2, i, sl])
            return c
        lax.fori_loop(0, STEP, addrow, 0, unroll=2)

    def fire_out(t, b):
        pltpu.async_copy(rows.at[b, 0],
                         out_hbm.at[pl.ds(obase + t * STEP, STEP)],
                         sem_out.at[b])

    def drain_out(b):
        pltpu.make_async_copy(rows.at[b, 0],
                              out_hbm.at[pl.ds(obase, STEP)],
                              sem_out.at[b]).wait()

    for t in range(NBUF):
        fire_idx(t, t)
    for t in range(2):
        drain_idx(t)
        fire_gather(t)

    def body4(u, carry):
        t0 = NBUF * u
        for b in range(NBUF):
            t = t0 + b

            @pl.when(t < NSTEPS - 2)
            def _():
                @pl.when(t >= 2)
                def _():
                    drain_out((b + 2) % NBUF)
                drain_idx((b + 2) % NBUF)
                fire_gather((b + 2) % NBUF)

            drain_gather(b)

            @pl.when(t < NSTEPS - NBUF)
            def _():
                fire_idx(t + NBUF, b)

            accum(b)
            fire_out(t, b)
        return carry

    lax.fori_loop(0, NSTEPS // NBUF, body4, 0)
    for b in range(NBUF):
        drain_out(b)


@functools.partial(jax.jit, static_argnames=())
def _emb_call(idx0, idx1, idx2, W0, W1, W2):
    mesh = plsc.VectorSubcoreMesh(core_axis_name="c", subcore_axis_name="s")
    return pl.kernel(
        _emb_body,
        out_type=jax.ShapeDtypeStruct((N, D), jnp.float32),
        mesh=mesh,
        scratch_types=[
            pltpu.VMEM((NBUF, 3, STEP), jnp.int32),
            pltpu.VMEM((NBUF, 3, STEP, D), jnp.float32),
            pltpu.SemaphoreType.DMA((NBUF,)),
            pltpu.SemaphoreType.DMA((NBUF,)),
            pltpu.SemaphoreType.DMA((NBUF,)),
        ],
        compiler_params=pltpu.CompilerParams(use_tc_tiling_on_sc=False),
    )(idx0, idx1, idx2, W0, W1, W2)


def kernel(factored_tokens, W0, W1, W2):
    ft = factored_tokens.reshape(N, 3).astype(jnp.int32)
    idx = ft.T.reshape(3, NROWS, STEP)
    out = _emb_call(idx[0], idx[1], idx[2], W0, W1, W2)
    return out.reshape(B, L, D)
